# Initial kernel scaffold; baseline (speedup 1.0000x reference)
import jax
import jax.numpy as jnp
from jax.experimental import pallas as pl


def kernel(ids, table):
    raise NotImplementedError("write your pallas kernel here")


# speedup vs baseline: 2.6960x; 2.6960x over previous
import functools

import jax
import jax.numpy as jnp
from jax import lax
from jax.experimental import pallas as pl
from jax.experimental.pallas import tpu as pltpu
from jax.experimental.pallas import tpu_sc as plsc

B = 16384
L = 50
EMB = 32
NW = 32
BPW = B // NW
NBUF = 4
HALF = 16

_mesh = plsc.VectorSubcoreMesh(core_axis_name="c", subcore_axis_name="s")


@functools.partial(
    pl.kernel,
    out_type=jax.ShapeDtypeStruct((B, EMB), jnp.float32),
    mesh=_mesh,
    scratch_types=[
        pltpu.VMEM((BPW, L), jnp.int32),
        pltpu.VMEM((NBUF, L, EMB), jnp.float32),
        pltpu.VMEM((BPW, EMB), jnp.float32),
    ] + [pltpu.SemaphoreType.DMA] * NBUF,
    compiler_params=pltpu.CompilerParams(use_tc_tiling_on_sc=False),
)
def _lookup(ids_hbm, table_hbm, out_hbm, idx_v, rows_v, out_v, *sems):
    wid = lax.axis_index("s") * 2 + lax.axis_index("c")
    pltpu.sync_copy(ids_hbm.at[wid], idx_v)

    def _fire(e, b):
        return pltpu.async_copy(table_hbm.at[idx_v.at[e]], rows_v.at[b], sems[b])

    def _wait(e, b):
        pltpu.make_async_copy(table_hbm.at[idx_v.at[e]], rows_v.at[b], sems[b]).wait()

    for b in range(NBUF):
        _fire(b, b)

    def body(g, carry):
        for b in range(NBUF):
            e = g * NBUF + b
            _wait(e, b)
            acc0 = rows_v[b, 0, pl.ds(0, HALF)]
            acc1 = rows_v[b, 0, pl.ds(HALF, HALF)]
            for j in range(1, L):
                acc0 = acc0 + rows_v[b, j, pl.ds(0, HALF)]
                acc1 = acc1 + rows_v[b, j, pl.ds(HALF, HALF)]
            nxt = e + NBUF

            @pl.when(nxt < BPW)
            def _():
                _fire(nxt, b)

            out_v[e, pl.ds(0, HALF)] = acc0 * (1.0 / L)
            out_v[e, pl.ds(HALF, HALF)] = acc1 * (1.0 / L)
        return carry

    lax.fori_loop(0, BPW // NBUF, body, 0)
    pltpu.sync_copy(out_v, out_hbm.at[pl.ds(wid * BPW, BPW)])


def kernel(ids, table):
    return _lookup(ids.reshape(NW, BPW, L), table)
